# baseline (speedup 1.0000x reference)
import jax
import jax.numpy as jnp
from jax.experimental import pallas as pl
from jax.experimental.pallas import tpu as pltpu

_B, _N, _D = 2048, 16, 64
_BB = 16
_GRID = _B // _BB


def _lrelu(x):
    return jnp.where(x >= 0, x, 0.01 * x)


def _pad8(w):
    return jnp.pad(w, ((0, 8 - w.shape[0]), (0, 0)))


def _body(flat_ref, We1_ref, be1_ref, We2_ref, be2_ref, Wh1h_ref, Wh1t2_ref,
          Wh1s_ref, bh1_ref, whard_ref, bhard_ref, Wq_ref, Wkt_ref, Wks_ref,
          Wvt_ref, Wvs_ref, bv_ref, Ws1q_ref, Ws1k_ref, bs1_ref, ws2_ref,
          bs2_ref, Wd1h_ref, Wd1o_ref, bd1_ref, Wd2_ref, bd2_ref, out_ref):
    f32 = jnp.float32
    dot = lambda a, b: jnp.dot(a, b, preferred_element_type=f32)
    flat = flat_ref[...]

    h = _lrelu(dot(flat, We1_ref[...]) + be1_ref[...])
    h = _lrelu(dot(h, We2_ref[...]) + be2_ref[...])

    tpart = dot(h, Wh1h_ref[...]) - dot(flat, Wh1t2_ref[...]) + bh1_ref[...]
    spart = dot(flat, Wh1s_ref[...])
    kt = dot(flat, Wkt_ref[...])
    ks = dot(flat, Wks_ref[...])
    vt = dot(flat, Wvt_ref[...])
    vs = dot(flat, Wvs_ref[...])
    qs1 = dot(dot(h, Wq_ref[...]), Ws1q_ref[...]) + bs1_ref[...]

    def n3(x):
        return x.reshape(_BB, _N, x.shape[-1])

    hh = jnp.maximum(n3(tpart)[:, :, None, :] + n3(spart)[:, None, :, :], 0.0)
    hard_logit = jnp.sum(hh * whard_ref[...].reshape(1, 1, 1, _D), axis=-1)
    hard = jax.nn.sigmoid(hard_logit + bhard_ref[0, 0])

    k4 = _lrelu(n3(kt)[:, :, None, :] + n3(ks)[:, None, :, :])
    sk = dot(k4.reshape(_BB * _N * _N, _D), Ws1k_ref[...])
    spre = jnp.maximum(sk.reshape(_BB, _N, _N, _D)
                       + n3(qs1)[:, :, None, :], 0.0)
    scores = jnp.sum(spre * ws2_ref[...].reshape(1, 1, 1, _D), axis=-1)
    scores = scores + bs2_ref[0, 0]

    ti = jax.lax.broadcasted_iota(jnp.int32, (_N, _N), 0)
    si = jax.lax.broadcasted_iota(jnp.int32, (_N, _N), 1)
    scores = jnp.where((ti == si)[None, :, :], -1e30, scores)
    m = jnp.max(scores, axis=2, keepdims=True)
    ex = jnp.exp(scores - m)
    w = ex / jnp.sum(ex, axis=2, keepdims=True) * hard

    v4 = _lrelu(n3(vt)[:, :, None, :] + n3(vs)[:, None, :, :]
                + bv_ref[...].reshape(1, 1, 1, _D))
    agg = jnp.sum(v4 * w[:, :, :, None], axis=2)

    dec = _lrelu(dot(h, Wd1h_ref[...])
                 + dot(agg.reshape(_BB * _N, _D), Wd1o_ref[...])
                 + bd1_ref[...])
    dec = _lrelu(dot(dec, Wd2_ref[...]) + bd2_ref[...])
    out_ref[...] = dec


def kernel(embedding, W_e1, b_e1, W_e2, b_e2, W_h1, b_h1, W_h2, b_h2,
           W_he, b_he, W_q, W_k, W_v, b_v, W_s1, b_s1, W_s2, b_s2,
           W_d1, b_d1, W_d2, b_d2):
    flatp = jnp.pad(embedding.reshape(_B * _N, 5), ((0, 0), (0, 3)))

    whe_diff = W_he[:, 1] - W_he[:, 0]
    w_hard = (W_h2 @ whe_diff).reshape(1, _D)
    b_hard = (b_h2 @ whe_diff + b_he[1] - b_he[0]).reshape(1, 1)
    weights = (
        _pad8(W_e1), b_e1.reshape(1, -1), W_e2, b_e2.reshape(1, -1),
        W_h1[:_D], _pad8(W_h1[_D + 5:]),
        _pad8(W_h1[_D:_D + 5]) + _pad8(W_h1[_D + 5:]), b_h1.reshape(1, -1),
        w_hard, b_hard, W_q,
        _pad8(W_k[:5]), _pad8(W_k[5:]),
        _pad8(W_v[:5]), _pad8(W_v[5:]), b_v.reshape(1, -1),
        W_s1[:_D], W_s1[_D:], b_s1.reshape(1, -1),
        W_s2.reshape(1, _D), b_s2.reshape(1, 1),
        W_d1[:_D], W_d1[_D:], b_d1.reshape(1, -1),
        W_d2, b_d2.reshape(1, -1),
    )
    in_specs = [pl.BlockSpec((_BB * _N, 8), lambda i: (i, 0))] + [
        pl.BlockSpec(w.shape, lambda i: (0, 0)) for w in weights
    ]
    return pl.pallas_call(
        _body,
        grid=(_GRID,),
        in_specs=in_specs,
        out_specs=pl.BlockSpec((_BB * _N, 2 * _D), lambda i: (i, 0)),
        out_shape=jax.ShapeDtypeStruct((_B * _N, 2 * _D), jnp.float32),
        compiler_params=pltpu.CompilerParams(
            dimension_semantics=("parallel",)),
    )(flatp, *weights)


# speedup vs baseline: 1.6152x; 1.6152x over previous
import jax
import jax.numpy as jnp
import numpy as np
from jax.experimental import pallas as pl
from jax.experimental.pallas import tpu as pltpu

_B, _N, _D = 2048, 16, 64
_G = 16
_L = _G * _N
_GRID = _B // _G


def _lrelu(x):
    return jnp.where(x >= 0, x, 0.01 * x)


def _pad8(w):
    return jnp.pad(w, ((0, 8 - w.shape[0]), (0, 0)))


def _body(flat_ref, frep_ref, We1_ref, be1_ref, We2_ref, be2_ref, Wh1h_ref,
          Wh1t2_ref, Wh1s_ref, bh1_ref, whard_ref, bhard_ref, Wq_ref,
          Wkt_ref, Wks_ref, Wvt_ref, Wvs_ref, bv_ref, Ws1q_ref, Ws1k_ref,
          bs1_ref, ws2_ref, bs2_ref, Wd1h_ref, Wd1o_ref, bd1_ref, Wd2_ref,
          bd2_ref, out_ref):
    f32 = jnp.float32
    dot = lambda a, b: jnp.dot(a, b, preferred_element_type=f32)
    flat = flat_ref[0]

    h1 = _lrelu(dot(We1_ref[...], flat) + be1_ref[...])
    h = _lrelu(dot(We2_ref[...], h1) + be2_ref[...])

    tpart = dot(Wh1h_ref[...], h) - dot(Wh1t2_ref[...], flat) + bh1_ref[...]
    kt = dot(Wkt_ref[...], flat)
    vt = dot(Wvt_ref[...], flat) + bv_ref[...]
    qs1 = dot(Ws1q_ref[...], dot(Wq_ref[...], h)) + bs1_ref[...]

    frep = frep_ref[0]
    sp_rep = dot(Wh1s_ref[...], frep)
    ks_rep = dot(Wks_ref[...], frep)
    vs_rep = dot(Wvs_ref[...], frep)

    whard = whard_ref[...]
    ws2 = ws2_ref[...]
    lane_t = jax.lax.broadcasted_iota(jnp.int32, (1, _L), 1) % _N

    hard_rows, score_rows, v4s = [], [], []
    for s in range(_N):
        sl = slice(s * _L, (s + 1) * _L)
        hh = jnp.maximum(tpart + sp_rep[:, sl], 0.0)
        hard_rows.append(jnp.sum(hh * whard, axis=0, keepdims=True))
        k4 = _lrelu(kt + ks_rep[:, sl])
        spre = jnp.maximum(dot(Ws1k_ref[...], k4) + qs1, 0.0)
        sc = jnp.sum(spre * ws2, axis=0, keepdims=True) + bs2_ref[0, 0]
        score_rows.append(jnp.where(lane_t == s, -1e30, sc))
        v4s.append(_lrelu(vt + vs_rep[:, sl]))

    scores = jnp.concatenate(score_rows, axis=0)
    m = jnp.max(scores, axis=0, keepdims=True)
    ex = jnp.exp(scores - m)
    hard_logit = jnp.concatenate(hard_rows, axis=0) + bhard_ref[0, 0]
    w = (ex / jnp.sum(ex, axis=0, keepdims=True)) * jax.nn.sigmoid(hard_logit)

    agg = w[0:1] * v4s[0]
    for s in range(1, _N):
        agg = agg + w[s:s + 1] * v4s[s]

    dec = _lrelu(dot(Wd1h_ref[...], h) + dot(Wd1o_ref[...], agg)
                 + bd1_ref[...])
    dec = _lrelu(dot(Wd2_ref[...], dec) + bd2_ref[...])
    out_ref[...] = dec.T


def kernel(embedding, W_e1, b_e1, W_e2, b_e2, W_h1, b_h1, W_h2, b_h2,
           W_he, b_he, W_q, W_k, W_v, b_v, W_s1, b_s1, W_s2, b_s2,
           W_d1, b_d1, W_d2, b_d2):
    flat_pad = jnp.pad(embedding.reshape(_B * _N, 5), ((0, 0), (0, 3)))
    flat_t = flat_pad.T.reshape(1, 8, _B * _N)

    frep = jnp.broadcast_to(
        flat_pad.reshape(_GRID, _G, _N, 8).transpose(0, 3, 2, 1)
        [:, :, :, :, None],
        (_GRID, 8, _N, _G, _N)).reshape(_GRID, 8, _N * _L)

    whe_diff = W_he[:, 1] - W_he[:, 0]
    col1 = lambda b: b.reshape(-1, 1)
    weights = (
        _pad8(W_e1).T, col1(b_e1), W_e2.T, col1(b_e2),
        W_h1[:_D].T, _pad8(W_h1[_D + 5:]).T,
        (_pad8(W_h1[_D:_D + 5]) + _pad8(W_h1[_D + 5:])).T, col1(b_h1),
        (W_h2 @ whe_diff).reshape(_D, 1),
        (b_h2 @ whe_diff + b_he[1] - b_he[0]).reshape(1, 1),
        W_q.T,
        _pad8(W_k[:5]).T, _pad8(W_k[5:]).T,
        _pad8(W_v[:5]).T, _pad8(W_v[5:]).T, col1(b_v),
        W_s1[:_D].T, W_s1[_D:].T, col1(b_s1),
        W_s2.reshape(_D, 1), b_s2.reshape(1, 1),
        W_d1[:_D].T, W_d1[_D:].T, col1(b_d1),
        W_d2.T, col1(b_d2),
    )
    in_specs = [pl.BlockSpec((1, 8, _L), lambda i: (0, 0, i)),
                pl.BlockSpec((1, 8, _N * _L), lambda i: (i, 0, 0))] + [
        pl.BlockSpec(w.shape, lambda i: (0, 0)) for w in weights
    ]
    return pl.pallas_call(
        _body,
        grid=(_GRID,),
        in_specs=in_specs,
        out_specs=pl.BlockSpec((_L, 2 * _D), lambda i: (i, 0)),
        out_shape=jax.ShapeDtypeStruct((_B * _N, 2 * _D), jnp.float32),
        compiler_params=pltpu.CompilerParams(
            dimension_semantics=("parallel",)),
    )(flat_t, frep, *weights)


# speedup vs baseline: 1.8129x; 1.1224x over previous
import jax
import jax.numpy as jnp
import numpy as np
from jax.experimental import pallas as pl
from jax.experimental.pallas import tpu as pltpu

_B, _N, _D = 2048, 16, 64
_G = 64
_L = _G * _N
_GRID = _B // _G


def _lrelu(x):
    return jnp.maximum(x, 0.01 * x)


def _pad8(w):
    return jnp.pad(w, ((0, 8 - w.shape[0]), (0, 0)))


def _body(flat_ref, frep_ref, We1_ref, be1_ref, We2_ref, be2_ref, Wh1h_ref,
          Wh1t2_ref, Wh1s_ref, bh1_ref, whard_ref, bhard_ref, Wq_ref,
          Wkt_ref, Wks_ref, Wvt_ref, Wvs_ref, bv_ref, Ws1q_ref, Ws1k_ref,
          bs1_ref, ws2_ref, bs2_ref, Wd1h_ref, Wd1o_ref, bd1_ref, Wd2_ref,
          bd2_ref, out_ref):
    f32 = jnp.float32
    dot = lambda a, b: jnp.dot(a, b, preferred_element_type=f32)
    flat = flat_ref[0]

    h1 = _lrelu(dot(We1_ref[...], flat) + be1_ref[...])
    h = _lrelu(dot(We2_ref[...], h1) + be2_ref[...])

    tpart = dot(Wh1h_ref[...], h) - dot(Wh1t2_ref[...], flat) + bh1_ref[...]
    kt = dot(Wkt_ref[...], flat)
    vt = dot(Wvt_ref[...], flat) + bv_ref[...]
    qs1 = dot(Ws1q_ref[...], dot(Wq_ref[...], h)) + bs1_ref[...]

    frep = frep_ref[0]
    sp_rep = dot(Wh1s_ref[...], frep)
    ks_rep = dot(Wks_ref[...], frep)
    vs_rep = dot(Wvs_ref[...], frep)

    whard = whard_ref[...]
    ws2 = ws2_ref[...]
    lane_t = jax.lax.broadcasted_iota(jnp.int32, (1, _L), 1) % _N

    hard_rows, score_rows, v4s = [], [], []
    for s in range(_N):
        sl = slice(s * _L, (s + 1) * _L)
        hh = jnp.maximum(tpart + sp_rep[:, sl], 0.0)
        hard_rows.append(jnp.sum(hh * whard, axis=0, keepdims=True))
        k4 = _lrelu(kt + ks_rep[:, sl])
        spre = jnp.maximum(dot(Ws1k_ref[...], k4) + qs1, 0.0)
        sc = jnp.sum(spre * ws2, axis=0, keepdims=True) + bs2_ref[0, 0]
        score_rows.append(jnp.where(lane_t == s, -1e30, sc))
        v4s.append(_lrelu(vt + vs_rep[:, sl]))

    scores = jnp.concatenate(score_rows, axis=0)
    m = jnp.max(scores, axis=0, keepdims=True)
    ex = jnp.exp(scores - m)
    hard_logit = jnp.concatenate(hard_rows, axis=0) + bhard_ref[0, 0]
    w = (ex / jnp.sum(ex, axis=0, keepdims=True)) * jax.nn.sigmoid(hard_logit)

    agg = w[0:1] * v4s[0]
    for s in range(1, _N):
        agg = agg + w[s:s + 1] * v4s[s]

    dec = _lrelu(dot(Wd1h_ref[...], h) + dot(Wd1o_ref[...], agg)
                 + bd1_ref[...])
    dec = _lrelu(dot(Wd2_ref[...], dec) + bd2_ref[...])
    out_ref[...] = dec.T


def kernel(embedding, W_e1, b_e1, W_e2, b_e2, W_h1, b_h1, W_h2, b_h2,
           W_he, b_he, W_q, W_k, W_v, b_v, W_s1, b_s1, W_s2, b_s2,
           W_d1, b_d1, W_d2, b_d2):
    flat_pad = jnp.pad(embedding.reshape(_B * _N, 5), ((0, 0), (0, 3)))
    flat_t = flat_pad.T.reshape(1, 8, _B * _N)

    frep = jnp.broadcast_to(
        flat_pad.reshape(_GRID, _G, _N, 8).transpose(0, 3, 2, 1)
        [:, :, :, :, None],
        (_GRID, 8, _N, _G, _N)).reshape(_GRID, 8, _N * _L)

    whe_diff = W_he[:, 1] - W_he[:, 0]
    col1 = lambda b: b.reshape(-1, 1)
    weights = (
        _pad8(W_e1).T, col1(b_e1), W_e2.T, col1(b_e2),
        W_h1[:_D].T, _pad8(W_h1[_D + 5:]).T,
        (_pad8(W_h1[_D:_D + 5]) + _pad8(W_h1[_D + 5:])).T, col1(b_h1),
        (W_h2 @ whe_diff).reshape(_D, 1),
        (b_h2 @ whe_diff + b_he[1] - b_he[0]).reshape(1, 1),
        W_q.T,
        _pad8(W_k[:5]).T, _pad8(W_k[5:]).T,
        _pad8(W_v[:5]).T, _pad8(W_v[5:]).T, col1(b_v),
        W_s1[:_D].T, W_s1[_D:].T, col1(b_s1),
        W_s2.reshape(_D, 1), b_s2.reshape(1, 1),
        W_d1[:_D].T, W_d1[_D:].T, col1(b_d1),
        W_d2.T, col1(b_d2),
    )
    in_specs = [pl.BlockSpec((1, 8, _L), lambda i: (0, 0, i)),
                pl.BlockSpec((1, 8, _N * _L), lambda i: (i, 0, 0))] + [
        pl.BlockSpec(w.shape, lambda i: (0, 0)) for w in weights
    ]
    return pl.pallas_call(
        _body,
        grid=(_GRID,),
        in_specs=in_specs,
        out_specs=pl.BlockSpec((_L, 2 * _D), lambda i: (i, 0)),
        out_shape=jax.ShapeDtypeStruct((_B * _N, 2 * _D), jnp.float32),
        compiler_params=pltpu.CompilerParams(
            dimension_semantics=("parallel",)),
    )(flat_t, frep, *weights)


# speedup vs baseline: 2.3611x; 1.3024x over previous
import jax
import jax.numpy as jnp
import numpy as np
from jax.experimental import pallas as pl
from jax.experimental.pallas import tpu as pltpu

_B, _N, _D = 2048, 16, 64
_G = 64
_L = _G * _N
_GRID = _B // _G


def _lrelu(x):
    return jnp.maximum(x, 0.01 * x)


def _pad8(w):
    return jnp.pad(w, ((0, 8 - w.shape[0]), (0, 0)))


def _body(flat_ref, We1_ref, be1_ref, We2_ref, be2_ref, Wh1h_ref,
          Wh1t2_ref, Wh1s_ref, bh1_ref, whard_ref, bhard_ref, Wq_ref,
          Wkt_ref, Wks_ref, Wvt_ref, Wvs_ref, bv_ref, Ws1q_ref, Ws1k_ref,
          bs1_ref, ws2_ref, bs2_ref, Wd1h_ref, Wd1o_ref, bd1_ref, Wd2_ref,
          bd2_ref, out_ref):
    f32 = jnp.float32
    dot = lambda a, b: jnp.dot(a, b, preferred_element_type=f32)
    flat = flat_ref[0]

    h1 = _lrelu(dot(We1_ref[...], flat) + be1_ref[...])
    h = _lrelu(dot(We2_ref[...], h1) + be2_ref[...])

    tpart = dot(Wh1h_ref[...], h) - dot(Wh1t2_ref[...], flat) + bh1_ref[...]
    kt = dot(Wkt_ref[...], flat)
    vt = dot(Wvt_ref[...], flat) + bv_ref[...]
    qs1 = dot(Ws1q_ref[...], dot(Wq_ref[...], h)) + bs1_ref[...]

    spart = dot(Wh1s_ref[...], flat)
    ks = dot(Wks_ref[...], flat)
    vs = dot(Wvs_ref[...], flat)

    whard = whard_ref[...]
    ws2 = ws2_ref[...]
    lane_t = jax.lax.broadcasted_iota(jnp.int32, (1, _L), 1) // _G

    hard_rows, score_rows, v4s = [], [], []
    for s in range(_N):
        st = slice(s * _G, (s + 1) * _G)
        hh = jnp.maximum(tpart + pltpu.repeat(spart[:, st], _N, 1), 0.0)
        hard_rows.append(jnp.sum(hh * whard, axis=0, keepdims=True))
        k4 = _lrelu(kt + pltpu.repeat(ks[:, st], _N, 1))
        spre = jnp.maximum(dot(Ws1k_ref[...], k4) + qs1, 0.0)
        sc = jnp.sum(spre * ws2, axis=0, keepdims=True) + bs2_ref[0, 0]
        score_rows.append(jnp.where(lane_t == s, -1e30, sc))
        v4s.append(_lrelu(vt + pltpu.repeat(vs[:, st], _N, 1)))

    scores = jnp.concatenate(score_rows, axis=0)
    m = jnp.max(scores, axis=0, keepdims=True)
    ex = jnp.exp(scores - m)
    hard_logit = jnp.concatenate(hard_rows, axis=0) + bhard_ref[0, 0]
    w = (ex / jnp.sum(ex, axis=0, keepdims=True)) * jax.nn.sigmoid(hard_logit)

    agg = w[0:1] * v4s[0]
    for s in range(1, _N):
        agg = agg + w[s:s + 1] * v4s[s]

    dec = _lrelu(dot(Wd1h_ref[...], h) + dot(Wd1o_ref[...], agg)
                 + bd1_ref[...])
    dec = _lrelu(dot(Wd2_ref[...], dec) + bd2_ref[...])
    out_ref[0] = dec.T.reshape(_N, _G, 2 * _D).swapaxes(0, 1)


def kernel(embedding, W_e1, b_e1, W_e2, b_e2, W_h1, b_h1, W_h2, b_h2,
           W_he, b_he, W_q, W_k, W_v, b_v, W_s1, b_s1, W_s2, b_s2,
           W_d1, b_d1, W_d2, b_d2):
    flat_pad = jnp.pad(embedding.reshape(_B * _N, 5), ((0, 0), (0, 3)))
    flat_t = (flat_pad.reshape(_GRID, _G, _N, 8)
              .transpose(0, 3, 2, 1).reshape(_GRID, 8, _L))

    whe_diff = W_he[:, 1] - W_he[:, 0]
    col1 = lambda b: b.reshape(-1, 1)
    weights = (
        _pad8(W_e1).T, col1(b_e1), W_e2.T, col1(b_e2),
        W_h1[:_D].T, _pad8(W_h1[_D + 5:]).T,
        (_pad8(W_h1[_D:_D + 5]) + _pad8(W_h1[_D + 5:])).T, col1(b_h1),
        (W_h2 @ whe_diff).reshape(_D, 1),
        (b_h2 @ whe_diff + b_he[1] - b_he[0]).reshape(1, 1),
        W_q.T,
        _pad8(W_k[:5]).T, _pad8(W_k[5:]).T,
        _pad8(W_v[:5]).T, _pad8(W_v[5:]).T, col1(b_v),
        W_s1[:_D].T, W_s1[_D:].T, col1(b_s1),
        W_s2.reshape(_D, 1), b_s2.reshape(1, 1),
        W_d1[:_D].T, W_d1[_D:].T, col1(b_d1),
        W_d2.T, col1(b_d2),
    )
    in_specs = [pl.BlockSpec((1, 8, _L), lambda i: (i, 0, 0))] + [
        pl.BlockSpec(w.shape, lambda i: (0, 0)) for w in weights
    ]
    out = pl.pallas_call(
        _body,
        grid=(_GRID,),
        in_specs=in_specs,
        out_specs=pl.BlockSpec((1, _G, _N, 2 * _D), lambda i: (i, 0, 0, 0)),
        out_shape=jax.ShapeDtypeStruct((_GRID, _G, _N, 2 * _D), jnp.float32),
        compiler_params=pltpu.CompilerParams(
            dimension_semantics=("parallel",)),
    )(flat_t, *weights)
    return out.reshape(_B * _N, 2 * _D)
